# baseline (speedup 1.0000x reference)
import dataclasses
import functools

import jax
import jax.numpy as jnp
from jax import lax
from jax.experimental import pallas as pl
from jax.experimental.pallas import tpu as pltpu
from jax.experimental.pallas import tpu_sc as plsc

N_BINS = 15
NSLOTS = 16
LANES = 16
NW = 32
CHUNK = 8192


def _sc_hist(conf_hbm, acc_hbm, out_hbm, cnt_ref, csum_ref, asum_ref):
    zeros = jnp.zeros((LANES,), jnp.float32)
    for r in range(NSLOTS):
        cnt_ref[r, :] = zeros
        csum_ref[r, :] = zeros
        asum_ref[r, :] = zeros

    lanes = lax.iota(jnp.int32, LANES)
    ones = jnp.ones((LANES,), jnp.float32)

    def body(c_vm, a_vm):
        @pl.loop(0, CHUNK, step=LANES)
        def _(j):
            c = c_vm[pl.ds(j, LANES)]
            a = a_vm[pl.ds(j, LANES)]
            b = (c * jnp.float32(N_BINS)).astype(jnp.int32)
            b = jnp.clip(b, 0, NSLOTS - 1)
            plsc.addupdate_scatter(cnt_ref, [b, lanes], ones)
            plsc.addupdate_scatter(csum_ref, [b, lanes], c)
            plsc.addupdate_scatter(asum_ref, [b, lanes], a)

    n = conf_hbm.shape[0]
    pltpu.emit_pipeline(
        body,
        grid=(n // CHUNK,),
        in_specs=[
            pl.BlockSpec((CHUNK,), lambda i: (i,)),
            pl.BlockSpec((CHUNK,), lambda i: (i,)),
        ],
        out_specs=[],
        core_axis_name=("c", "s"),
        dimension_semantics=(pltpu.PARALLEL,),
    )(conf_hbm, acc_hbm)

    wid = lax.axis_index("c") * 16 + lax.axis_index("s")
    pltpu.sync_copy(cnt_ref, out_hbm.at[0, wid])
    pltpu.sync_copy(csum_ref, out_hbm.at[1, wid])
    pltpu.sync_copy(asum_ref, out_hbm.at[2, wid])


def _finish_body(n_total, p_ref, o_ref):
    p = p_ref[...]
    tot = jnp.sum(p, axis=(1, 3))
    cnt = tot[0:1, 0:N_BINS]
    csum = tot[1:2, 0:N_BINS]
    asum = tot[2:3, 0:N_BINS]
    safe = jnp.maximum(cnt, 1.0)
    diff = (csum - asum) / safe
    contrib = diff * diff * (cnt / jnp.float32(n_total))
    contrib = jnp.where(cnt > 0, contrib, 0.0)
    o_ref[...] = jnp.sum(contrib, axis=(0, 1), keepdims=True)


def kernel(confidences, accuracies):
    n = confidences.shape[0]
    mesh = plsc.VectorSubcoreMesh(core_axis_name="c", subcore_axis_name="s")
    cp = pltpu.CompilerParams()
    if "needs_layout_passes" in pltpu.CompilerParams.__dataclass_fields__:
        cp = dataclasses.replace(cp, needs_layout_passes=False)
    hist = pl.kernel(
        _sc_hist,
        out_type=jax.ShapeDtypeStruct((3, NW, NSLOTS, LANES), jnp.float32),
        mesh=mesh,
        scratch_types=[
            pltpu.VMEM((NSLOTS, LANES), jnp.float32),
            pltpu.VMEM((NSLOTS, LANES), jnp.float32),
            pltpu.VMEM((NSLOTS, LANES), jnp.float32),
        ],
        compiler_params=cp,
    )(confidences, accuracies)
    out = pl.pallas_call(
        functools.partial(_finish_body, n),
        out_shape=jax.ShapeDtypeStruct((1, 1), jnp.float32),
    )(hist)
    return out[0, 0]


# speedup vs baseline: 3.4664x; 3.4664x over previous
import dataclasses
import functools

import jax
import jax.numpy as jnp
from jax import lax
from jax.experimental import pallas as pl
from jax.experimental.pallas import tpu as pltpu
from jax.experimental.pallas import tpu_sc as plsc

N_BINS = 15
NSLOTS = 16
LANES = 16
NW = 32
CHUNK = 16384


def _sc_hist(conf_hbm, acc_hbm, out_hbm, cnt_ref, csum_ref, asum_ref):
    zeros = jnp.zeros((LANES,), jnp.float32)
    for r in range(NSLOTS):
        cnt_ref[r, :] = zeros
        csum_ref[r, :] = zeros
        asum_ref[r, :] = zeros

    lanes = lax.iota(jnp.int32, LANES)
    ones = jnp.ones((LANES,), jnp.float32)

    def body(c_vm, a_vm):
        @plsc.parallel_loop(0, CHUNK, step=LANES, unroll=8)
        def _(j):
            c = c_vm[pl.ds(j, LANES)]
            a = a_vm[pl.ds(j, LANES)]
            b = (c * jnp.float32(N_BINS)).astype(jnp.int32)
            b = jnp.clip(b, 0, NSLOTS - 1)
            plsc.addupdate_scatter(cnt_ref, [b, lanes], ones)
            plsc.addupdate_scatter(csum_ref, [b, lanes], c)
            plsc.addupdate_scatter(asum_ref, [b, lanes], a)

    n = conf_hbm.shape[0]
    pltpu.emit_pipeline(
        body,
        grid=(n // CHUNK,),
        in_specs=[
            pl.BlockSpec((CHUNK,), lambda i: (i,)),
            pl.BlockSpec((CHUNK,), lambda i: (i,)),
        ],
        out_specs=[],
        core_axis_name=("c", "s"),
        dimension_semantics=(pltpu.PARALLEL,),
    )(conf_hbm, acc_hbm)

    wid = lax.axis_index("c") * 16 + lax.axis_index("s")
    pltpu.sync_copy(cnt_ref, out_hbm.at[0, wid])
    pltpu.sync_copy(csum_ref, out_hbm.at[1, wid])
    pltpu.sync_copy(asum_ref, out_hbm.at[2, wid])


def _finish_body(n_total, p_ref, o_ref):
    p = p_ref[...]
    tot = jnp.sum(p, axis=(1, 3))
    cnt = tot[0:1, 0:N_BINS]
    csum = tot[1:2, 0:N_BINS]
    asum = tot[2:3, 0:N_BINS]
    safe = jnp.maximum(cnt, 1.0)
    diff = (csum - asum) / safe
    contrib = diff * diff * (cnt / jnp.float32(n_total))
    contrib = jnp.where(cnt > 0, contrib, 0.0)
    o_ref[...] = jnp.sum(contrib, axis=(0, 1), keepdims=True)


def kernel(confidences, accuracies):
    n = confidences.shape[0]
    mesh = plsc.VectorSubcoreMesh(core_axis_name="c", subcore_axis_name="s")
    cp = pltpu.CompilerParams()
    if "needs_layout_passes" in pltpu.CompilerParams.__dataclass_fields__:
        cp = dataclasses.replace(cp, needs_layout_passes=False)
    hist = pl.kernel(
        _sc_hist,
        out_type=jax.ShapeDtypeStruct((3, NW, NSLOTS, LANES), jnp.float32),
        mesh=mesh,
        scratch_types=[
            pltpu.VMEM((NSLOTS, LANES), jnp.float32),
            pltpu.VMEM((NSLOTS, LANES), jnp.float32),
            pltpu.VMEM((NSLOTS, LANES), jnp.float32),
        ],
        compiler_params=cp,
    )(confidences, accuracies)
    out = pl.pallas_call(
        functools.partial(_finish_body, n),
        out_shape=jax.ShapeDtypeStruct((1, 1), jnp.float32),
    )(hist)
    return out[0, 0]


# speedup vs baseline: 3.5552x; 1.0256x over previous
import dataclasses
import functools

import jax
import jax.numpy as jnp
from jax import lax
from jax.experimental import pallas as pl
from jax.experimental.pallas import tpu as pltpu
from jax.experimental.pallas import tpu_sc as plsc

N_BINS = 15
NSLOTS = 16
LANES = 16
NW = 32
CHUNK = 16384


def _sc_hist(conf_hbm, acc_hbm, out_hbm, cnt_ref, csum_ref, asum_ref):
    zeros = jnp.zeros((LANES,), jnp.float32)
    for r in range(NSLOTS):
        cnt_ref[r, :] = zeros
        csum_ref[r, :] = zeros
        asum_ref[r, :] = zeros

    lanes = lax.iota(jnp.int32, LANES)
    ones = jnp.ones((LANES,), jnp.float32)

    def body(c_vm, a_vm):
        @plsc.parallel_loop(0, CHUNK, step=LANES, unroll=16)
        def _(j):
            c = c_vm[pl.ds(j, LANES)]
            a = a_vm[pl.ds(j, LANES)]
            b = (c * jnp.float32(N_BINS)).astype(jnp.int32)
            b = jnp.clip(b, 0, NSLOTS - 1)
            plsc.addupdate_scatter(cnt_ref, [b, lanes], ones)
            plsc.addupdate_scatter(csum_ref, [b, lanes], c)
            plsc.addupdate_scatter(asum_ref, [b, lanes], a)

    n = conf_hbm.shape[0]
    pltpu.emit_pipeline(
        body,
        grid=(n // CHUNK,),
        in_specs=[
            pl.BlockSpec((CHUNK,), lambda i: (i,)),
            pl.BlockSpec((CHUNK,), lambda i: (i,)),
        ],
        out_specs=[],
        core_axis_name=("c", "s"),
        dimension_semantics=(pltpu.PARALLEL,),
    )(conf_hbm, acc_hbm)

    wid = lax.axis_index("c") * 16 + lax.axis_index("s")
    pltpu.sync_copy(cnt_ref, out_hbm.at[0, wid])
    pltpu.sync_copy(csum_ref, out_hbm.at[1, wid])
    pltpu.sync_copy(asum_ref, out_hbm.at[2, wid])


def _finish_body(n_total, p_ref, o_ref):
    p = p_ref[...]
    tot = jnp.sum(p, axis=(1, 3))
    cnt = tot[0:1, 0:N_BINS]
    csum = tot[1:2, 0:N_BINS]
    asum = tot[2:3, 0:N_BINS]
    safe = jnp.maximum(cnt, 1.0)
    diff = (csum - asum) / safe
    contrib = diff * diff * (cnt / jnp.float32(n_total))
    contrib = jnp.where(cnt > 0, contrib, 0.0)
    o_ref[...] = jnp.sum(contrib, axis=(0, 1), keepdims=True)


def kernel(confidences, accuracies):
    n = confidences.shape[0]
    mesh = plsc.VectorSubcoreMesh(core_axis_name="c", subcore_axis_name="s")
    cp = pltpu.CompilerParams()
    if "needs_layout_passes" in pltpu.CompilerParams.__dataclass_fields__:
        cp = dataclasses.replace(cp, needs_layout_passes=False)
    hist = pl.kernel(
        _sc_hist,
        out_type=jax.ShapeDtypeStruct((3, NW, NSLOTS, LANES), jnp.float32),
        mesh=mesh,
        scratch_types=[
            pltpu.VMEM((NSLOTS, LANES), jnp.float32),
            pltpu.VMEM((NSLOTS, LANES), jnp.float32),
            pltpu.VMEM((NSLOTS, LANES), jnp.float32),
        ],
        compiler_params=cp,
    )(confidences, accuracies)
    out = pl.pallas_call(
        functools.partial(_finish_body, n),
        out_shape=jax.ShapeDtypeStruct((1, 1), jnp.float32),
    )(hist)
    return out[0, 0]
